# baseline (speedup 1.0000x reference)
import functools

import jax
import jax.numpy as jnp
from jax import lax
from jax.experimental import pallas as pl
from jax.experimental.pallas import tpu as pltpu
from jax.experimental.pallas import tpu_sc as plsc

F = 32
L = 16
RW = 128
CB = 16
GR = 80


@functools.cache
def _build(B, H, N):
    info = plsc.get_sparse_core_info()
    NC, NS = info.num_cores, info.num_subcores
    NW = NC * NS
    assert B % (NW * CB) == 0
    BPW = B // NW
    n_chunks = BPW // CB
    HB = CB * H // 2

    mesh = plsc.VectorSubcoreMesh(core_axis_name="c", subcore_axis_name="s")

    @functools.partial(
        pl.kernel,
        mesh=mesh,
        compiler_params=pltpu.CompilerParams(
            needs_layout_passes=False, use_tc_tiling_on_sc=True),
        out_type=jax.ShapeDtypeStruct((B,), jnp.float32),
        scratch_types=[
            pltpu.VMEM((CB,), jnp.int32),
            pltpu.VMEM((CB,), jnp.int32),
            pltpu.VMEM((CB,), jnp.int32),
            pltpu.VMEM((CB,), jnp.int32),
            pltpu.VMEM((CB,), jnp.int32),
            pltpu.VMEM((CB,), jnp.int32),
            pltpu.VMEM((CB * H,), jnp.int32),
            pltpu.VMEM((CB * H,), jnp.int32),
            pltpu.VMEM((CB * H,), jnp.int32),
            pltpu.VMEM((CB, RW), jnp.float32),
            pltpu.VMEM((CB, RW), jnp.float32),
            pltpu.VMEM((HB, RW), jnp.float32),
            pltpu.VMEM((HB, RW), jnp.float32),
            pltpu.VMEM((CB, L), jnp.float32),
            pltpu.VMEM((BPW,), jnp.float32),
            pltpu.SemaphoreType.DMA,
        ],
    )
    def _k(th_h, be_h, rh_h, us_h, it_h, cx_h, out_h,
           uidx, iidx, uline, uqo, iline, iqo, cidx, cline, cqo,
           th_v, be_v, rho_a, rho_b, q_v, outb, sem):
        wid = lax.axis_index("s") * NC + lax.axis_index("c")
        base = pl.multiple_of(wid * BPW, BPW)

        def chunk_body(c, carry):
            gb = pl.multiple_of(base + c * CB, CB)
            gbc = pl.multiple_of((base + c * CB) * H, CB * H)
            pltpu.sync_copy(us_h.at[pl.ds(gb, CB)], uidx)
            pltpu.sync_copy(it_h.at[pl.ds(gb, CB)], iidx)
            pltpu.sync_copy(cx_h.at[pl.ds(gbc, CB * H)], cidx)

            v = uidx[...]
            uline[...] = v >> 2
            uqo[...] = (v & 3) << 5
            v = iidx[...]
            iline[...] = v >> 2
            iqo[...] = (v & 3) << 5

            def split_body(t, carry2):
                off = pl.multiple_of(t * L, L)
                v2 = cidx[pl.ds(off, L)]
                cline[pl.ds(off, L)] = v2 >> 2
                cqo[pl.ds(off, L)] = (v2 & 3) << 5
                return carry2

            lax.fori_loop(0, CB * H // L, split_body, 0)

            cps = [
                pltpu.async_copy(th_h.at[uline], th_v, sem),
                pltpu.async_copy(be_h.at[iline], be_v, sem),
            ]
            for g in range(0, HB, GR):
                cps.append(pltpu.async_copy(
                    rh_h.at[cline.at[pl.ds(g, GR)]],
                    rho_a.at[pl.ds(g, GR)], sem))
                cps.append(pltpu.async_copy(
                    rh_h.at[cline.at[pl.ds(HB + g, GR)]],
                    rho_b.at[pl.ds(HB + g - HB, GR)], sem))
            for cp in cps:
                cp.wait()

            lanes_i = lax.iota(jnp.int32, L)

            def splat(x):
                return jnp.full((L,), x, jnp.int32)

            def make_e_body(buf, half):
                def e_body(e, carry2):
                    ge = e + half * (CB // 2)
                    ge_s = splat(ge)
                    uo = plsc.load_gather(uqo, [ge_s])
                    acc0 = plsc.load_gather(th_v, [ge_s, uo + lanes_i])
                    acc1 = plsc.load_gather(th_v, [ge_s, uo + lanes_i + L])
                    for h in range(H):
                        lrow = splat(e * H + h)
                        o = plsc.load_gather(cqo, [splat(half * HB) + lrow])
                        cols = o + lanes_i
                        acc0 = acc0 + plsc.load_gather(buf, [lrow, cols])
                        acc1 = acc1 + plsc.load_gather(buf, [lrow, cols + L])
                    io = plsc.load_gather(iqo, [ge_s])
                    b0 = plsc.load_gather(be_v, [ge_s, io + lanes_i])
                    b1 = plsc.load_gather(be_v, [ge_s, io + lanes_i + L])
                    q_v[ge, pl.ds(0, L)] = b0 * acc0 + b1 * acc1
                    return carry2
                return e_body

            lax.fori_loop(0, CB // 2, make_e_body(rho_a, 0), 0)
            lax.fori_loop(0, CB // 2, make_e_body(rho_b, 1), 0)

            lanes = lax.iota(jnp.int32, L)
            svec = jnp.zeros((L,), jnp.float32)
            for j in range(L):
                svec = svec + plsc.load_gather(
                    q_v, [lanes, jnp.full((L,), j, jnp.int32)])
            outb[pl.ds(pl.multiple_of(c * CB, CB), CB)] = svec
            return carry

        lax.fori_loop(0, n_chunks, chunk_body, 0)

        def sig_body(i, carry):
            off = pl.multiple_of(i * L, L)
            x = outb[pl.ds(off, L)]
            outb[pl.ds(off, L)] = 1.0 / (1.0 + jnp.exp(-x))
            return carry

        lax.fori_loop(0, BPW // L, sig_body, 0)
        pltpu.sync_copy(outb, out_h.at[pl.ds(base, BPW)])

    return _k


def kernel(theta, beta, rho, user, item, contexts):
    B, H = contexts.shape
    N = theta.shape[0]
    return _build(B, H, N)(
        theta.reshape(N // 4, RW), beta.reshape(N // 4, RW),
        rho.reshape(N // 4, RW), user, item, contexts.reshape(B * H))
